# baseline (speedup 1.0000x reference)
import functools

import jax
import jax.numpy as jnp
import numpy as np
from jax import lax
from jax.experimental import pallas as pl
from jax.experimental.pallas import tpu as pltpu
from jax.experimental.pallas import tpu_sc as plsc

B = 16384
D = 32
NUM_F = 5
_SIZES = (11, 12, 31, 24, 6)
_CUM = tuple(int(v) for v in np.concatenate([[0], np.cumsum(_SIZES)[:-1]]))
V_ALL = sum(_SIZES)

NC, NS, L = 2, 16, 16
NW = NC * NS
RW = B // NW
PW = RW * NUM_F
XR, XC = 8, PW // 8
VPR = XC // L

_OFF_PHASES = np.array(
    [[_CUM[(phase + l) % NUM_F] for l in range(L)] for phase in range(NUM_F)],
    dtype=np.int32,
)


def _body(x_hbm, off_hbm, tall_hbm, out_hbm, xv, offv, tallv, outv, sem):
    wid = lax.axis_index("s") * NC + lax.axis_index("c")
    pltpu.sync_copy(x_hbm.at[wid], xv)
    pltpu.sync_copy(off_hbm, offv)
    pltpu.sync_copy(tall_hbm, tallv)
    st_iota = lax.iota(jnp.int32, L) * D

    def x_row(r, _):
        base = r * (XC * D)
        for k in range(VPR):
            xvec = xv[r, pl.ds(k * L, L)]
            ib = (xvec + offv[k % NUM_F, :]) * D
            st = st_iota + (base + k * L * D)
            for h in range(0, D, 16):
                vals = [plsc.load_gather(tallv, [ib + j]) for j in range(h, h + 16)]
                for j in range(16):
                    plsc.store_scatter(outv, [st + h + j], vals[j])
        rows = XC // NUM_F
        pltpu.make_async_copy(
            outv.at[pl.ds(base, XC * D)],
            out_hbm.at[pl.ds((wid * XR + r) * rows * NUM_F * D, XC * D)],
            sem,
        ).start()
        return ()

    lax.fori_loop(0, XR, x_row, (), unroll=False)
    for r in range(XR):
        pltpu.make_async_copy(
            outv.at[pl.ds(r * XC * D, XC * D)],
            out_hbm.at[pl.ds((wid * XR + r) * XC * D, XC * D)],
            sem,
        ).wait()


@jax.jit
def _embed(x3, off, tall):
    mesh = plsc.VectorSubcoreMesh(core_axis_name="c", subcore_axis_name="s")
    run = functools.partial(
        pl.kernel,
        mesh=mesh,
        out_type=jax.ShapeDtypeStruct((B * NUM_F * D,), jnp.float32),
        scratch_types=[
            pltpu.VMEM((XR, XC), jnp.int32),
            pltpu.VMEM((NUM_F, L), jnp.int32),
            pltpu.VMEM((V_ALL * D,), jnp.float32),
            pltpu.VMEM((PW * D,), jnp.float32),
            pltpu.SemaphoreType.DMA,
        ],
        compiler_params=pltpu.CompilerParams(
            use_tc_tiling_on_sc=False, needs_layout_passes=False
        ),
    )(_body)
    return run(x3, off, tall)


def kernel(x, table_year, table_month, table_day, table_hour, table_weekday):
    tall = jnp.concatenate(
        [table_year, table_month, table_day, table_hour, table_weekday], axis=0
    ).reshape(-1)
    x3 = x.astype(jnp.int32).reshape(NW, XR, XC)
    out = _embed(x3, jnp.asarray(_OFF_PHASES), tall)
    return out.reshape(B, NUM_F * D)


# speedup vs baseline: 1.4560x; 1.4560x over previous
import functools

import jax
import jax.numpy as jnp
import numpy as np
from jax import lax
from jax.experimental import pallas as pl
from jax.experimental.pallas import tpu as pltpu
from jax.experimental.pallas import tpu_sc as plsc

B = 16384
D = 32
DP = D + 1
NUM_F = 5
OD = NUM_F * D
OP = OD + 5
_SIZES = (11, 12, 31, 24, 6)
_CUM = tuple(int(v) for v in np.concatenate([[0], np.cumsum(_SIZES)[:-1]]))
V_ALL = sum(_SIZES)

NC, NS, L = 2, 16, 16
NW = NC * NS
RW = B // NW
SW = RW * NUM_F
NCH = 8
CR = RW // NCH
GC = CR // L


def _body(x_hbm, tall_hbm, out_hbm, xv, tallv, outv, sem):
    wid = lax.axis_index("s") * NC + lax.axis_index("c")
    pltpu.sync_copy(x_hbm.at[wid], xv)
    pltpu.sync_copy(tall_hbm, tallv)
    iota = lax.iota(jnp.int32, L)
    zero = iota - iota
    iota5 = iota * NUM_F

    def chunk(c, _):
        for g in range(GC):
            rowv = iota + (c * CR + g * L)
            for f in range(NUM_F):
                xvec = plsc.load_gather(
                    xv, [iota5 + (c * (CR * NUM_F) + g * (L * NUM_F) + f)]
                )
                grow = xvec + _CUM[f]
                la = 5
                vals = []
                for j in range(D):
                    vals.append(plsc.load_gather(tallv, [grow, zero + j]))
                    if j >= la:
                        plsc.store_scatter(
                            outv, [rowv, zero + (f * D + j - la)], vals[j - la]
                        )
                for j in range(D - la, D):
                    plsc.store_scatter(outv, [rowv, zero + (f * D + j)], vals[j])
        pltpu.make_async_copy(
            outv.at[pl.ds(c * CR, CR), pl.ds(0, OD)],
            out_hbm.at[pl.ds(wid * RW + c * CR, CR)],
            sem,
        ).start()
        return ()

    lax.fori_loop(0, NCH, chunk, (), unroll=False)
    for c in range(NCH):
        pltpu.make_async_copy(
            outv.at[pl.ds(c * CR, CR), pl.ds(0, OD)],
            out_hbm.at[pl.ds(wid * RW + c * CR, CR)],
            sem,
        ).wait()


@jax.jit
def _embed(x2, tall):
    mesh = plsc.VectorSubcoreMesh(core_axis_name="c", subcore_axis_name="s")
    run = functools.partial(
        pl.kernel,
        mesh=mesh,
        out_type=jax.ShapeDtypeStruct((B, OD), jnp.float32),
        scratch_types=[
            pltpu.VMEM((SW,), jnp.int32),
            pltpu.VMEM((V_ALL, DP), jnp.float32),
            pltpu.VMEM((RW, OP), jnp.float32),
            pltpu.SemaphoreType.DMA,
        ],
        compiler_params=pltpu.CompilerParams(
            use_tc_tiling_on_sc=False, needs_layout_passes=False
        ),
    )(_body)
    return run(x2, tall)


def kernel(x, table_year, table_month, table_day, table_hour, table_weekday):
    tall = jnp.concatenate(
        [table_year, table_month, table_day, table_hour, table_weekday], axis=0
    )
    tall = jnp.pad(tall, ((0, 0), (0, DP - D)))
    x2 = x.astype(jnp.int32).reshape(NW, SW)
    return _embed(x2, tall)


# speedup vs baseline: 1.7853x; 1.2262x over previous
import functools

import jax
import jax.numpy as jnp
import numpy as np
from jax import lax
from jax.experimental import pallas as pl
from jax.experimental.pallas import tpu as pltpu
from jax.experimental.pallas import tpu_sc as plsc

B = 16384
D = 32
NUM_F = 5
_SIZES = (11, 12, 31, 24, 6)
_CUM = tuple(int(v) for v in np.concatenate([[0], np.cumsum(_SIZES)[:-1]]))
V_ALL = sum(_SIZES)

NC, NS, L = 2, 16, 16
NW = NC * NS
PW = B * NUM_F // NW
CHUNK = 128
NCHUNK = PW // CHUNK

_OFF_PHASES = np.array(
    [[_CUM[(phase + l) % NUM_F] for l in range(L)] for phase in range(NUM_F)],
    dtype=np.int32,
)


def _body(x_hbm, off_hbm, tall_hbm, out_hbm, xv, offv, tallv, idxv, outv, sem):
    wid = lax.axis_index("s") * NC + lax.axis_index("c")
    pltpu.sync_copy(x_hbm.at[wid], xv)
    pltpu.sync_copy(off_hbm, offv)
    sid = lax.axis_index("s")

    @pl.when(sid == 0)
    def _():
        pltpu.sync_copy(tall_hbm, tallv)

    plsc.subcore_barrier()
    vregs_per_row = CHUNK // L
    for i in range(NCHUNK):
        for j in range(vregs_per_row):
            v = i * vregs_per_row + j
            off = offv[v % NUM_F, :]
            sl = pl.ds(j * L, L)
            idxv[i, sl] = xv[i, sl] + off
    copies = [
        pltpu.make_async_copy(
            tallv.at[idxv.at[i]],
            outv.at[pl.ds(i * CHUNK, CHUNK)],
            sem,
        )
        for i in range(NCHUNK)
    ]
    for c in copies:
        c.start()
    for c in copies:
        c.wait()
    pltpu.sync_copy(outv, out_hbm.at[pl.ds(wid * PW, PW)])


@jax.jit
def _embed(x3, off, tall):
    mesh = plsc.VectorSubcoreMesh(core_axis_name="c", subcore_axis_name="s")
    run = functools.partial(
        pl.kernel,
        mesh=mesh,
        out_type=jax.ShapeDtypeStruct((B * NUM_F, D), jnp.float32),
        scratch_types=[
            pltpu.VMEM((NCHUNK, CHUNK), jnp.int32),
            pltpu.VMEM((NUM_F, L), jnp.int32),
            pltpu.VMEM_SHARED((V_ALL, D), jnp.float32),
            pltpu.VMEM((NCHUNK, CHUNK), jnp.int32),
            pltpu.VMEM((PW, D), jnp.float32),
            pltpu.SemaphoreType.DMA,
        ],
        compiler_params=pltpu.CompilerParams(use_tc_tiling_on_sc=False),
    )(_body)
    return run(x3, off, tall)


def kernel(x, table_year, table_month, table_day, table_hour, table_weekday):
    tall = jnp.concatenate(
        [table_year, table_month, table_day, table_hour, table_weekday], axis=0
    )
    x3 = x.astype(jnp.int32).reshape(NW, NCHUNK, CHUNK)
    out = _embed(x3, jnp.asarray(_OFF_PHASES), tall)
    return out.reshape(B, NUM_F * D)


# speedup vs baseline: 1.8392x; 1.0302x over previous
import functools

import jax
import jax.numpy as jnp
import numpy as np
from jax import lax
from jax.experimental import pallas as pl
from jax.experimental.pallas import tpu as pltpu
from jax.experimental.pallas import tpu_sc as plsc

B = 16384
D = 32
NUM_F = 5
_SIZES = (11, 12, 31, 24, 6)
_CUM = tuple(int(v) for v in np.concatenate([[0], np.cumsum(_SIZES)[:-1]]))
V_ALL = sum(_SIZES)

NC, NS, L = 2, 16, 16
NW = NC * NS
PW = B * NUM_F // NW
CHUNK = 128
NCHUNK = PW // CHUNK
NSEM = 4

_OFF_PHASES = np.array(
    [[_CUM[(phase + l) % NUM_F] for l in range(L)] for phase in range(NUM_F)],
    dtype=np.int32,
)


def _body(x_hbm, off_hbm, tall_hbm, out_hbm, xv, offv, tallv, idxv, outv, *sems):
    gsem = sems[:NSEM]
    osem = sems[NSEM]
    wid = lax.axis_index("s") * NC + lax.axis_index("c")
    pltpu.sync_copy(x_hbm.at[wid], xv)
    pltpu.sync_copy(off_hbm, offv)
    sid = lax.axis_index("s")

    @pl.when(sid == 0)
    def _():
        pltpu.sync_copy(tall_hbm, tallv)

    gathers = [
        pltpu.make_async_copy(
            tallv.at[idxv.at[i]],
            outv.at[pl.ds(i * CHUNK, CHUNK)],
            gsem[i % NSEM],
        )
        for i in range(NCHUNK)
    ]
    stores = [
        pltpu.make_async_copy(
            outv.at[pl.ds(i * CHUNK, CHUNK)],
            out_hbm.at[pl.ds(wid * PW + i * CHUNK, CHUNK)],
            osem,
        )
        for i in range(NCHUNK)
    ]
    vregs_per_row = CHUNK // L
    for i in range(NCHUNK):
        for j in range(vregs_per_row):
            v = i * vregs_per_row + j
            off = offv[v % NUM_F, :]
            sl = pl.ds(j * L, L)
            idxv[i, sl] = xv[i, sl] + off
        if i == 0:
            plsc.subcore_barrier()
        if i >= NSEM:
            gathers[i - NSEM].wait()
            stores[i - NSEM].start()
        gathers[i].start()
    for i in range(NCHUNK - NSEM, NCHUNK):
        gathers[i].wait()
        stores[i].start()
    for s in stores:
        s.wait()


@jax.jit
def _embed(x3, off, tall):
    mesh = plsc.VectorSubcoreMesh(core_axis_name="c", subcore_axis_name="s")
    run = functools.partial(
        pl.kernel,
        mesh=mesh,
        out_type=jax.ShapeDtypeStruct((B * NUM_F, D), jnp.float32),
        scratch_types=[
            pltpu.VMEM((NCHUNK, CHUNK), jnp.int32),
            pltpu.VMEM((NUM_F, L), jnp.int32),
            pltpu.VMEM_SHARED((V_ALL, D), jnp.float32),
            pltpu.VMEM((NCHUNK, CHUNK), jnp.int32),
            pltpu.VMEM((PW, D), jnp.float32),
            pltpu.SemaphoreType.DMA,
            pltpu.SemaphoreType.DMA,
            pltpu.SemaphoreType.DMA,
            pltpu.SemaphoreType.DMA,
            pltpu.SemaphoreType.DMA,
        ],
        compiler_params=pltpu.CompilerParams(use_tc_tiling_on_sc=False),
    )(_body)
    return run(x3, off, tall)


def kernel(x, table_year, table_month, table_day, table_hour, table_weekday):
    tall = jnp.concatenate(
        [table_year, table_month, table_day, table_hour, table_weekday], axis=0
    )
    x3 = x.astype(jnp.int32).reshape(NW, NCHUNK, CHUNK)
    out = _embed(x3, jnp.asarray(_OFF_PHASES), tall)
    return out.reshape(B, NUM_F * D)
